# baseline (speedup 1.0000x reference)
import functools

import jax
import jax.numpy as jnp
from jax import lax
from jax.experimental import pallas as pl
from jax.experimental.pallas import tpu as pltpu
from jax.experimental.pallas import tpu_sc as plsc

K = 16
CHUNK = 16384


def _sc_kernel(rows, length, nw):
    chunks_per_row = length // CHUNK
    rows_per_w = rows // nw
    n_chunks = rows_per_w * chunks_per_row
    win_per_chunk = CHUNK // K
    mesh = plsc.VectorSubcoreMesh(core_axis_name="c", subcore_axis_name="s")

    @functools.partial(
        pl.kernel,
        out_type=jax.ShapeDtypeStruct((rows, length), jnp.float32),
        mesh=mesh,
        scratch_types=[
            pltpu.VMEM((CHUNK,), jnp.float32),
            pltpu.VMEM((CHUNK,), jnp.float32),
        ],
    )
    def k(x_hbm, o_hbm, in_v, out_v):
        wid = lax.axis_index("s") * 2 + lax.axis_index("c")
        iota = lax.iota(jnp.int32, K)
        rots = [(iota + s) & (K - 1) for s in (1, 2, 4, 8)]

        def win_body(w, _):
            v = in_v[pl.ds(w * K, K)]
            a = jnp.abs(v)
            m = a
            for r in rots:
                m = jnp.maximum(m, m.at[r].get(mode="promise_in_bounds"))
            ci = jnp.where(a >= m, iota, K)
            for r in rots:
                ci = jnp.minimum(ci, ci.at[r].get(mode="promise_in_bounds"))
            out_v[pl.ds(w * K, K)] = jnp.where(iota == ci, v, 0.0)
            return 0

        def chunk_body(i, _):
            row = wid * rows_per_w + i // chunks_per_row
            c0 = (i % chunks_per_row) * CHUNK
            pltpu.sync_copy(x_hbm.at[row, pl.ds(c0, CHUNK)], in_v)
            lax.fori_loop(0, win_per_chunk, win_body, 0, unroll=8)
            pltpu.sync_copy(out_v, o_hbm.at[row, pl.ds(c0, CHUNK)])
            return 0

        lax.fori_loop(0, n_chunks, chunk_body, 0)

    return k


def kernel(input_):
    b, c, l = input_.shape
    rows = b * c
    info = plsc.get_sparse_core_info()
    nw = info.num_cores * info.num_subcores
    x2 = input_.reshape(rows, l)
    out = _sc_kernel(rows, l, nw)(x2)
    return out.reshape(b, c, l)


# speedup vs baseline: 1.0815x; 1.0815x over previous
import functools

import jax
import jax.numpy as jnp
from jax import lax
from jax.experimental import pallas as pl
from jax.experimental.pallas import tpu as pltpu
from jax.experimental.pallas import tpu_sc as plsc

K = 16
CHUNK = 16384


def _sc_kernel(rows, length, nw):
    chunks_per_row = length // CHUNK
    rows_per_w = rows // nw
    n_chunks = rows_per_w * chunks_per_row
    n_pairs = n_chunks // 2
    win_per_chunk = CHUNK // K
    mesh = plsc.VectorSubcoreMesh(core_axis_name="c", subcore_axis_name="s")

    @functools.partial(
        pl.kernel,
        out_type=jax.ShapeDtypeStruct((rows, length), jnp.float32),
        mesh=mesh,
        scratch_types=[
            pltpu.VMEM((CHUNK,), jnp.float32),
            pltpu.VMEM((CHUNK,), jnp.float32),
            pltpu.VMEM((CHUNK,), jnp.float32),
            pltpu.VMEM((CHUNK,), jnp.float32),
            pltpu.SemaphoreType.DMA,
            pltpu.SemaphoreType.DMA,
            pltpu.SemaphoreType.DMA,
            pltpu.SemaphoreType.DMA,
        ],
    )
    def k(x_hbm, o_hbm, in0, in1, ou0, ou1, si0, si1, so0, so1):
        wid = lax.axis_index("s") * 2 + lax.axis_index("c")
        iota = lax.iota(jnp.int32, K)
        rots = [(iota + s) & (K - 1) for s in (1, 2, 4, 8)]

        def coords(i):
            row = wid * rows_per_w + i // chunks_per_row
            c0 = (i % chunks_per_row) * CHUNK
            return row, c0

        def start_in(i, buf, sem):
            row, c0 = coords(i)
            pltpu.async_copy(x_hbm.at[row, pl.ds(c0, CHUNK)], buf, sem)

        def wait_in(i, buf, sem):
            row, c0 = coords(i)
            pltpu.make_async_copy(
                x_hbm.at[row, pl.ds(c0, CHUNK)], buf, sem).wait()

        def start_out(i, buf, sem):
            row, c0 = coords(i)
            pltpu.async_copy(buf, o_hbm.at[row, pl.ds(c0, CHUNK)], sem)

        def wait_out(i, buf, sem):
            row, c0 = coords(i)
            pltpu.make_async_copy(
                buf, o_hbm.at[row, pl.ds(c0, CHUNK)], sem).wait()

        def compute(src, dst):
            def win_body(w, _):
                v = src[pl.ds(w * K, K)]
                a = jnp.abs(v)
                m = a
                for r in rots:
                    m = jnp.maximum(m, m.at[r].get(mode="promise_in_bounds"))
                ci = jnp.where(a >= m, iota, K)
                for r in rots:
                    ci = jnp.minimum(ci,
                                     ci.at[r].get(mode="promise_in_bounds"))
                dst[pl.ds(w * K, K)] = jnp.where(iota == ci, v, 0.0)
                return 0

            lax.fori_loop(0, win_per_chunk, win_body, 0, unroll=16)

        start_in(0, in0, si0)
        start_in(1, in1, si1)

        def pair_body(p, _):
            for b in (0, 1):
                ib, ob, sib, sob = ((in0, ou0, si0, so0),
                                    (in1, ou1, si1, so1))[b]
                i = 2 * p + b
                wait_in(i, ib, sib)

                @pl.when(p > 0)
                def _():
                    wait_out(i - 2, ob, sob)

                compute(ib, ob)
                start_out(i, ob, sob)

                @pl.when(p < n_pairs - 1)
                def _():
                    start_in(i + 2, ib, sib)
            return 0

        lax.fori_loop(0, n_pairs, pair_body, 0)
        wait_out(n_chunks - 2, ou0, so0)
        wait_out(n_chunks - 1, ou1, so1)

    return k


def kernel(input_):
    b, c, l = input_.shape
    rows = b * c
    info = plsc.get_sparse_core_info()
    nw = info.num_cores * info.num_subcores
    x2 = input_.reshape(rows, l)
    out = _sc_kernel(rows, l, nw)(x2)
    return out.reshape(b, c, l)


# speedup vs baseline: 2.6500x; 2.4503x over previous
import functools

import jax
import jax.numpy as jnp
from jax import lax
from jax.experimental import pallas as pl
from jax.experimental.pallas import tpu as pltpu
from jax.experimental.pallas import tpu_sc as plsc

K = 16
CHUNK = 16384


def _sc_kernel(rows, length, nw):
    chunks_per_row = length // CHUNK
    rows_per_w = rows // nw
    n_chunks = rows_per_w * chunks_per_row
    n_pairs = n_chunks // 2
    win_per_chunk = CHUNK // K
    mesh = plsc.VectorSubcoreMesh(core_axis_name="c", subcore_axis_name="s")

    @functools.partial(
        pl.kernel,
        out_type=jax.ShapeDtypeStruct((rows, length), jnp.float32),
        mesh=mesh,
        scratch_types=[
            pltpu.VMEM((CHUNK,), jnp.float32),
            pltpu.VMEM((CHUNK,), jnp.float32),
            pltpu.VMEM((CHUNK,), jnp.float32),
            pltpu.VMEM((CHUNK,), jnp.float32),
            pltpu.SemaphoreType.DMA,
            pltpu.SemaphoreType.DMA,
            pltpu.SemaphoreType.DMA,
            pltpu.SemaphoreType.DMA,
        ],
    )
    def k(x_hbm, o_hbm, in0, in1, ou0, ou1, si0, si1, so0, so1):
        wid = lax.axis_index("s") * 2 + lax.axis_index("c")
        iota = lax.iota(jnp.int32, K)
        rots = [(iota + s) & (K - 1) for s in (1, 2, 4, 8)]

        def coords(i):
            row = wid * rows_per_w + i // chunks_per_row
            c0 = (i % chunks_per_row) * CHUNK
            return row, c0

        def start_in(i, buf, sem):
            row, c0 = coords(i)
            pltpu.async_copy(x_hbm.at[row, pl.ds(c0, CHUNK)], buf, sem)

        def wait_in(i, buf, sem):
            row, c0 = coords(i)
            pltpu.make_async_copy(
                x_hbm.at[row, pl.ds(c0, CHUNK)], buf, sem).wait()

        def start_out(i, buf, sem):
            row, c0 = coords(i)
            pltpu.async_copy(buf, o_hbm.at[row, pl.ds(c0, CHUNK)], sem)

        def wait_out(i, buf, sem):
            row, c0 = coords(i)
            pltpu.make_async_copy(
                buf, o_hbm.at[row, pl.ds(c0, CHUNK)], sem).wait()

        def compute(src, dst):
            def win_body(w, _):
                v = src[pl.ds(w * K, K)]
                a = jnp.abs(v)
                m = a
                for r in rots:
                    m = jnp.maximum(m, m.at[r].get(mode="promise_in_bounds"))
                ci = jnp.where(a >= m, iota, K)
                for r in rots:
                    ci = jnp.minimum(ci,
                                     ci.at[r].get(mode="promise_in_bounds"))
                dst[pl.ds(w * K, K)] = jnp.where(iota == ci, v, 0.0)
                return 0

            lax.fori_loop(0, win_per_chunk, win_body, 0, unroll=16)

        start_in(0, in0, si0)
        start_in(1, in1, si1)

        def pair_body(p, _):
            for b in (0, 1):
                ib, ob, sib, sob = ((in0, ou0, si0, so0),
                                    (in1, ou1, si1, so1))[b]
                i = 2 * p + b
                wait_in(i, ib, sib)

                @pl.when(p > 0)
                def _():
                    wait_out(i - 2, ob, sob)

                compute(ib, ob)
                start_out(i, ob, sob)

                @pl.when(p < n_pairs - 1)
                def _():
                    start_in(i + 2, ib, sib)
            return 0

        lax.fori_loop(0, n_pairs, pair_body, 0)
        wait_out(n_chunks - 2, ou0, so0)
        wait_out(n_chunks - 1, ou1, so1)

    return k


TC_BLOCK_ROWS = 8192
SC_ROW_FRACTION = 0.25


def _tc_body(x_ref, o_ref, *, block_rows):
    lane = lax.broadcasted_iota(jnp.int32, (block_rows, 128), 1)
    lane16 = lane & (K - 1)
    j16f = lane16.astype(jnp.float32)

    ri = lax.broadcasted_iota(jnp.int32, (128, 128), 0)
    ci = lax.broadcasted_iota(jnp.int32, (128, 128), 1)
    bcast = jnp.where((ri % K == 0) & (ri // K == ci // K), 1.0, 0.0)

    c = x_ref[...]
    m = jnp.abs(c)
    idx = j16f
    for s in (1, 2, 4, 8):
        rm = jnp.roll(m, -s, axis=-1)
        rdx = jnp.roll(idx, -s, axis=-1)
        gt = rm > m
        m = jnp.where(gt, rm, m)
        idx = jnp.where(gt, rdx, idx)
    il = jnp.where(lane16 == 0, idx, 0.0)
    g = jnp.dot(il, bcast)
    o_ref[...] = jnp.where(g == j16f, c, 0.0)


def _tc_call(x2):
    rows, l = x2.shape
    n = rows * l // 128
    xv = x2.reshape(n, 128)
    br = min(TC_BLOCK_ROWS, n)
    out = pl.pallas_call(
        functools.partial(_tc_body, block_rows=br),
        grid=(n // br,),
        in_specs=[pl.BlockSpec((br, 128), lambda i: (i, 0))],
        out_specs=pl.BlockSpec((br, 128), lambda i: (i, 0)),
        out_shape=jax.ShapeDtypeStruct((n, 128), x2.dtype),
    )(xv)
    return out.reshape(rows, l)


def kernel(input_):
    b, c, l = input_.shape
    rows = b * c
    info = plsc.get_sparse_core_info()
    nw = info.num_cores * info.num_subcores
    x2 = input_.reshape(rows, l)
    sc_rows = int(rows * SC_ROW_FRACTION) // nw * nw
    out_sc = _sc_kernel(sc_rows, l, nw)(x2[:sc_rows])
    out_tc = _tc_call(x2[sc_rows:])
    out = jnp.concatenate([out_sc, out_tc], axis=0)
    return out.reshape(b, c, l)
